# baseline (speedup 1.0000x reference)
import jax
import jax.numpy as jnp
from jax import lax
from jax.experimental import pallas as pl
from jax.experimental.pallas import tpu as pltpu
from jax.experimental.pallas import tpu_sc as plsc

STEPS = 16
L = STEPS + 1
WIN = 8
N = 100000
DEG = 16
E = N * DEG
D = 128

NT = 32
W = 3200
NP = NT * W
CK = 64
CH = W // CK
RING = 4


def _mesh():
    return plsc.VectorSubcoreMesh(core_axis_name="c", subcore_axis_name="s",
                                  num_cores=2, num_subcores=16)


def _tile_base():
    cid = lax.axis_index("c")
    sid = lax.axis_index("s")
    return (sid * 2 + cid) * W


def _sc_walk_body(adjn_h, adjoff_h, deg_h, cho_h,
                  walks_h,
                  startb, rowa, rowb, degb, offb, chb, altb, newb, anewb,
                  gsem, wsem):
    base = _tile_base()

    def init_v(v, carry):
        ids = base + v * 16 + lax.iota(jnp.int32, 16)
        ids = jnp.minimum(ids, N - 1)
        startb[pl.ds(v * 16, 16)] = ids
        return carry
    lax.fori_loop(0, W // 16, init_v, 0)
    pltpu.async_copy(startb, walks_h.at[pl.ds(base, W)], wsem)

    def do_step(i, cur, prev, dst, with_bt, wait_dst_row):
        def cho_idx(v, carry):
            sl = pl.ds(v * 16, 16)
            chb[sl] = startb[sl] + i * N
            return carry
        lax.fori_loop(0, W // 16, cho_idx, 0)

        pltpu.async_copy(deg_h.at[cur], degb, gsem)
        pltpu.async_copy(adjoff_h.at[cur], offb, gsem)
        pltpu.async_copy(cho_h.at[chb], newb, gsem)
        pltpu.make_async_copy(deg_h.at[cur], degb, gsem).wait()
        pltpu.make_async_copy(adjoff_h.at[cur], offb, gsem).wait()
        pltpu.make_async_copy(cho_h.at[chb], newb, gsem).wait()

        def comp1(v, carry):
            sl = pl.ds(v * 16, 16)
            d = degb[sl]
            off = offb[sl]
            ch = newb[sl]
            ei = lax.rem(ch, d)
            nbd = jnp.maximum(d - 1, 1)
            ai = lax.rem(ei + 1 + lax.rem(ch, nbd), d)
            chb[sl] = off + ei
            altb[sl] = off + ai
            return carry
        lax.fori_loop(0, W // 16, comp1, 0)

        pltpu.async_copy(adjn_h.at[chb], newb, gsem)
        if with_bt:
            pltpu.async_copy(adjn_h.at[altb], anewb, gsem)
        pltpu.make_async_copy(adjn_h.at[chb], newb, gsem).wait()
        if with_bt:
            pltpu.make_async_copy(adjn_h.at[altb], anewb, gsem).wait()

        if wait_dst_row:
            pltpu.make_async_copy(
                dst, walks_h.at[pl.ds((i - 1) * NP + base, W)], wsem).wait()

        def comp2(v, carry):
            sl = pl.ds(v * 16, 16)
            nw = newb[sl]
            if with_bt:
                an = anewb[sl]
                pv = prev[sl]
                dst[sl] = jnp.where(nw == pv, an, nw)
            else:
                dst[sl] = nw
            return carry
        lax.fori_loop(0, W // 16, comp2, 0)

        pltpu.async_copy(dst, walks_h.at[pl.ds((i + 1) * NP + base, W)],
                         wsem)

    do_step(0, startb, None, rowb, False, False)
    do_step(1, rowb, startb, rowa, True, False)

    def pair(t, carry):
        do_step(2 * t, rowa, rowb, rowb, True, True)
        do_step(2 * t + 1, rowb, rowa, rowa, True, True)
        return carry
    lax.fori_loop(1, STEPS // 2, pair, 0)

    pltpu.make_async_copy(startb, walks_h.at[pl.ds(base, W)], wsem).wait()
    pltpu.make_async_copy(rowb, walks_h.at[pl.ds(15 * NP + base, W)],
                          wsem).wait()
    pltpu.make_async_copy(rowa, walks_h.at[pl.ds(16 * NP + base, W)],
                          wsem).wait()


def _sc_pool_body(x_h, walks_h,
                  pooled_h,
                  wk, ring, idxb, accS,
                  lsem, gs0, gs1, gs2, gs3, ss0, ss1, ss2, ss3, psem):
    base = _tile_base()
    sid = lax.axis_index("s")
    rstart = sid * CK
    gs = [gs0, gs1, gs2, gs3]
    ss = [ss0, ss1, ss2, ss3]

    def lin(j, carry):
        pltpu.async_copy(walks_h.at[pl.ds(j * NP + base, W)], wk.at[j], lsem)
        return carry
    lax.fori_loop(0, L, lin, 0)

    def idx_init(v, carry):
        idxb[pl.ds(v * 16, 16)] = rstart + v * 16 + lax.iota(jnp.int32, 16)
        return carry
    lax.fori_loop(0, CK // 16, idx_init, 0)

    def lwait(j, carry):
        pltpu.make_async_copy(walks_h.at[pl.ds(j * NP + base, W)],
                              wk.at[j], lsem).wait()
        return carry
    lax.fori_loop(0, L, lwait, 0)

    def chunkloop(c, carry):
        ck0 = pl.ds(c * CK, CK)
        for j in range(RING - 1):
            pltpu.async_copy(x_h.at[wk.at[j, ck0]], ring.at[j], gs[j])
        for j in range(L):
            b = j % RING
            pltpu.make_async_copy(x_h.at[wk.at[j, ck0]],
                                  ring.at[b], gs[b]).wait()
            pltpu.async_copy(ring.at[b], accS.at[idxb], ss[b],
                             add=(j > 0))
            if j == 0:
                pltpu.make_async_copy(ring.at[0], accS.at[idxb],
                                      ss[0]).wait()
            jn = j + RING - 1
            if jn < L:
                if j >= 2:
                    pb = (j - 1) % RING
                    pltpu.make_async_copy(ring.at[pb], accS.at[idxb],
                                          ss[pb]).wait()
                pltpu.async_copy(x_h.at[wk.at[jn, ck0]],
                                 ring.at[jn % RING], gs[jn % RING])
        for j in range(L - RING, L):
            b = j % RING
            pltpu.make_async_copy(ring.at[b], accS.at[idxb], ss[b]).wait()
        cpw = pltpu.async_copy(accS.at[pl.ds(rstart, CK), :],
                               pooled_h.at[pl.ds(base + c * CK, CK), :],
                               psem)
        cpw.wait()
        return carry
    lax.fori_loop(0, CH, chunkloop, 0)


def _sc_walk(adj_nodes, adj_offset, degrees, choices_flat):
    kfn = pl.kernel(
        _sc_walk_body,
        out_type=[jax.ShapeDtypeStruct((L * NP,), jnp.int32)],
        mesh=_mesh(),
        scratch_types=[
            pltpu.VMEM((W,), jnp.int32),
            pltpu.VMEM((W,), jnp.int32),
            pltpu.VMEM((W,), jnp.int32),
            pltpu.VMEM((W,), jnp.int32),
            pltpu.VMEM((W,), jnp.int32),
            pltpu.VMEM((W,), jnp.int32),
            pltpu.VMEM((W,), jnp.int32),
            pltpu.VMEM((W,), jnp.int32),
            pltpu.VMEM((W,), jnp.int32),
            pltpu.SemaphoreType.DMA,
            pltpu.SemaphoreType.DMA,
        ],
    )
    return kfn(adj_nodes, adj_offset, degrees, choices_flat)[0]


def _sc_pool(x, walks_flat):
    kfn = pl.kernel(
        _sc_pool_body,
        out_type=[jax.ShapeDtypeStruct((NP, D), jnp.float32)],
        mesh=_mesh(),
        scratch_types=[
            pltpu.VMEM((L, W), jnp.int32),
            pltpu.VMEM((RING, CK, D), jnp.float32),
            pltpu.VMEM((CK,), jnp.int32),
            pltpu.VMEM_SHARED((16 * CK, D), jnp.float32),
            pltpu.SemaphoreType.DMA,
            pltpu.SemaphoreType.DMA,
            pltpu.SemaphoreType.DMA,
            pltpu.SemaphoreType.DMA,
            pltpu.SemaphoreType.DMA,
            pltpu.SemaphoreType.DMA,
            pltpu.SemaphoreType.DMA,
            pltpu.SemaphoreType.DMA,
            pltpu.SemaphoreType.DMA,
            pltpu.SemaphoreType.DMA,
        ],
    )
    return kfn(x, walks_flat)[0]


def _scale_body(x_ref, o_ref):
    o_ref[...] = x_ref[...] * jnp.float32(1.0 / L)


def _scale_tc(x):
    BR = 2000
    return pl.pallas_call(
        _scale_body,
        grid=(N // BR,),
        in_specs=[pl.BlockSpec((BR, D), lambda i: (i, 0))],
        out_specs=pl.BlockSpec((BR, D), lambda i: (i, 0)),
        out_shape=jax.ShapeDtypeStruct((N, D), jnp.float32),
    )(x)


def _idenc_body(w_ref, o_ref):
    w = w_ref[...]
    bn = w.shape[1]
    rows = []
    for t in range(WIN):
        d = WIN - t
        eq = (w[d:, :] == w[:-d, :]).astype(jnp.int32)
        z = jnp.zeros((d, bn), jnp.int32)
        rows.append(jnp.concatenate([z, eq], axis=0))
    o_ref[...] = jnp.stack(rows, axis=1) != 0


def _idenc_tc(walks_pad):
    BN = 2048
    grid = (NP // BN,)
    return pl.pallas_call(
        _idenc_body,
        grid=grid,
        in_specs=[pl.BlockSpec((L, BN), lambda i: (0, i))],
        out_specs=pl.BlockSpec((L, WIN, BN), lambda i: (0, 0, i)),
        out_shape=jax.ShapeDtypeStruct((L, WIN, NP), jnp.bool_),
    )(walks_pad)


def kernel(x, adj_nodes, adj_offset, degrees, choices):
    xs = _scale_tc(x)
    walks_flat = _sc_walk(adj_nodes, adj_offset, degrees,
                          choices.reshape(-1))
    pooled_pad = _sc_pool(xs, walks_flat)
    walks_pad = walks_flat.reshape(L, NP)
    walks = walks_pad[:, :N]
    pooled = pooled_pad[:N]
    id_enc = _idenc_tc(walks_pad)[:, :, :N]
    return pooled, walks, id_enc


# speedup vs baseline: 2.0578x; 2.0578x over previous
import jax
import jax.numpy as jnp
from jax import lax
from jax.experimental import pallas as pl
from jax.experimental.pallas import tpu as pltpu
from jax.experimental.pallas import tpu_sc as plsc

STEPS = 16
L = STEPS + 1
WIN = 8
N = 100000
DEG = 16
E = N * DEG
D = 128

NT = 32
W = 3200
NP = NT * W
CK = 64
CH = W // CK
RING = 4


def _mesh():
    return plsc.VectorSubcoreMesh(core_axis_name="c", subcore_axis_name="s",
                                  num_cores=2, num_subcores=16)


def _tile_base():
    cid = lax.axis_index("c")
    sid = lax.axis_index("s")
    return (sid * 2 + cid) * W


def _sc_walk_body(adjn_h, pk_h, cho_h,
                  walks_h,
                  startb, rowa, rowb, pkb, chsb, chb, altb, newb, anewb,
                  gsem, wsem):
    base = _tile_base()

    def init_v(v, carry):
        ids = base + v * 16 + lax.iota(jnp.int32, 16)
        ids = jnp.minimum(ids, N - 1)
        startb[pl.ds(v * 16, 16)] = ids
        return carry
    lax.fori_loop(0, W // 16, init_v, 0)
    pltpu.async_copy(startb, walks_h.at[pl.ds(base, W)], wsem)

    def do_step(i, cur, prev, dst, with_bt, wait_dst_row):
        cho_sl = cho_h.at[pl.ds(i * N + base, W)]
        pltpu.async_copy(pk_h.at[cur], pkb, gsem)
        pltpu.async_copy(cho_sl, chsb, gsem)
        pltpu.make_async_copy(pk_h.at[cur], pkb, gsem).wait()
        pltpu.make_async_copy(cho_sl, chsb, gsem).wait()

        def comp1(v, carry):
            sl = pl.ds(v * 16, 16)
            pk = pkb[sl]
            d = lax.bitwise_and(pk, 31)
            off = lax.shift_right_logical(pk, 5)
            ch = chsb[sl]
            ei = lax.rem(ch, d)
            nbd = jnp.maximum(d - 1, 1)
            ai = lax.rem(ei + 1 + lax.rem(ch, nbd), d)
            chb[sl] = off + ei
            altb[sl] = off + ai
            return carry
        lax.fori_loop(0, W // 16, comp1, 0)

        pltpu.async_copy(adjn_h.at[chb], newb, gsem)
        if with_bt:
            pltpu.async_copy(adjn_h.at[altb], anewb, gsem)
        pltpu.make_async_copy(adjn_h.at[chb], newb, gsem).wait()
        if with_bt:
            pltpu.make_async_copy(adjn_h.at[altb], anewb, gsem).wait()

        if wait_dst_row:
            pltpu.make_async_copy(
                dst, walks_h.at[pl.ds((i - 1) * NP + base, W)], wsem).wait()

        def comp2(v, carry):
            sl = pl.ds(v * 16, 16)
            nw = newb[sl]
            if with_bt:
                an = anewb[sl]
                pv = prev[sl]
                dst[sl] = jnp.where(nw == pv, an, nw)
            else:
                dst[sl] = nw
            return carry
        lax.fori_loop(0, W // 16, comp2, 0)

        pltpu.async_copy(dst, walks_h.at[pl.ds((i + 1) * NP + base, W)],
                         wsem)

    do_step(0, startb, None, rowb, False, False)
    do_step(1, rowb, startb, rowa, True, False)

    def pair(t, carry):
        do_step(2 * t, rowa, rowb, rowb, True, True)
        do_step(2 * t + 1, rowb, rowa, rowa, True, True)
        return carry
    lax.fori_loop(1, STEPS // 2, pair, 0)

    pltpu.make_async_copy(startb, walks_h.at[pl.ds(base, W)], wsem).wait()
    pltpu.make_async_copy(rowb, walks_h.at[pl.ds(15 * NP + base, W)],
                          wsem).wait()
    pltpu.make_async_copy(rowa, walks_h.at[pl.ds(16 * NP + base, W)],
                          wsem).wait()


def _sc_pool_body(x_h, walks_h,
                  pooled_h,
                  wk, ring, idxb, accS,
                  lsem, gs0, gs1, gs2, gs3, ss0, ss1, ss2, ss3, psem):
    base = _tile_base()
    sid = lax.axis_index("s")
    rstart = sid * CK
    gs = [gs0, gs1, gs2, gs3]
    ss = [ss0, ss1, ss2, ss3]

    def lin(j, carry):
        pltpu.async_copy(walks_h.at[pl.ds(j * NP + base, W)], wk.at[j], lsem)
        return carry
    lax.fori_loop(0, L, lin, 0)

    def idx_init(v, carry):
        idxb[pl.ds(v * 16, 16)] = rstart + v * 16 + lax.iota(jnp.int32, 16)
        return carry
    lax.fori_loop(0, CK // 16, idx_init, 0)

    def lwait(j, carry):
        pltpu.make_async_copy(walks_h.at[pl.ds(j * NP + base, W)],
                              wk.at[j], lsem).wait()
        return carry
    lax.fori_loop(0, L, lwait, 0)

    def chunkloop(c, carry):
        ck0 = pl.ds(c * CK, CK)
        for j in range(RING - 1):
            pltpu.async_copy(x_h.at[wk.at[j, ck0]], ring.at[j], gs[j])
        for j in range(L):
            b = j % RING
            pltpu.make_async_copy(x_h.at[wk.at[j, ck0]],
                                  ring.at[b], gs[b]).wait()
            pltpu.async_copy(ring.at[b], accS.at[idxb], ss[b],
                             add=(j > 0))
            if j == 0:
                pltpu.make_async_copy(ring.at[0], accS.at[idxb],
                                      ss[0]).wait()
            jn = j + RING - 1
            if jn < L:
                if j >= 2:
                    pb = (j - 1) % RING
                    pltpu.make_async_copy(ring.at[pb], accS.at[idxb],
                                          ss[pb]).wait()
                pltpu.async_copy(x_h.at[wk.at[jn, ck0]],
                                 ring.at[jn % RING], gs[jn % RING])
        for j in range(L - RING, L):
            b = j % RING
            pltpu.make_async_copy(ring.at[b], accS.at[idxb], ss[b]).wait()
        cpw = pltpu.async_copy(accS.at[pl.ds(rstart, CK), :],
                               pooled_h.at[pl.ds(base + c * CK, CK), :],
                               psem)
        cpw.wait()
        return carry
    lax.fori_loop(0, CH, chunkloop, 0)


def _sc_walk(adj_nodes, packed, choices_flat):
    kfn = pl.kernel(
        _sc_walk_body,
        out_type=[jax.ShapeDtypeStruct((L * NP,), jnp.int32)],
        mesh=_mesh(),
        scratch_types=[
            pltpu.VMEM((W,), jnp.int32),
            pltpu.VMEM((W,), jnp.int32),
            pltpu.VMEM((W,), jnp.int32),
            pltpu.VMEM((W,), jnp.int32),
            pltpu.VMEM((W,), jnp.int32),
            pltpu.VMEM((W,), jnp.int32),
            pltpu.VMEM((W,), jnp.int32),
            pltpu.VMEM((W,), jnp.int32),
            pltpu.VMEM((W,), jnp.int32),
            pltpu.SemaphoreType.DMA,
            pltpu.SemaphoreType.DMA,
        ],
    )
    return kfn(adj_nodes, packed, choices_flat)[0]


def _sc_pool(x, walks_flat):
    kfn = pl.kernel(
        _sc_pool_body,
        out_type=[jax.ShapeDtypeStruct((NP, D), jnp.float32)],
        mesh=_mesh(),
        scratch_types=[
            pltpu.VMEM((L, W), jnp.int32),
            pltpu.VMEM((RING, CK, D), jnp.float32),
            pltpu.VMEM((CK,), jnp.int32),
            pltpu.VMEM_SHARED((16 * CK, D), jnp.float32),
            pltpu.SemaphoreType.DMA,
            pltpu.SemaphoreType.DMA,
            pltpu.SemaphoreType.DMA,
            pltpu.SemaphoreType.DMA,
            pltpu.SemaphoreType.DMA,
            pltpu.SemaphoreType.DMA,
            pltpu.SemaphoreType.DMA,
            pltpu.SemaphoreType.DMA,
            pltpu.SemaphoreType.DMA,
            pltpu.SemaphoreType.DMA,
        ],
    )
    return kfn(x, walks_flat)[0]


def _scale_body(x_ref, o_ref):
    o_ref[...] = x_ref[...] * jnp.float32(1.0 / L)


def _scale_tc(x):
    BR = 2000
    return pl.pallas_call(
        _scale_body,
        grid=(N // BR,),
        in_specs=[pl.BlockSpec((BR, D), lambda i: (i, 0))],
        out_specs=pl.BlockSpec((BR, D), lambda i: (i, 0)),
        out_shape=jax.ShapeDtypeStruct((N, D), jnp.float32),
    )(x)


def _idenc_body(w_ref, o_ref):
    w = w_ref[...]
    bn = w.shape[1]
    rows = []
    for t in range(WIN):
        d = WIN - t
        eq = (w[d:, :] == w[:-d, :]).astype(jnp.int32)
        z = jnp.zeros((d, bn), jnp.int32)
        rows.append(jnp.concatenate([z, eq], axis=0))
    o_ref[...] = jnp.stack(rows, axis=1) != 0


def _idenc_tc(walks_pad):
    BN = 2048
    grid = (NP // BN,)
    return pl.pallas_call(
        _idenc_body,
        grid=grid,
        in_specs=[pl.BlockSpec((L, BN), lambda i: (0, i))],
        out_specs=pl.BlockSpec((L, WIN, BN), lambda i: (0, 0, i)),
        out_shape=jax.ShapeDtypeStruct((L, WIN, NP), jnp.bool_),
    )(walks_pad)


def kernel(x, adj_nodes, adj_offset, degrees, choices):
    xs = _scale_tc(x)
    packed = adj_offset * 32 + degrees
    walks_flat = _sc_walk(adj_nodes, packed, choices.reshape(-1))
    pooled_pad = _sc_pool(xs, walks_flat)
    walks_pad = walks_flat.reshape(L, NP)
    walks = walks_pad[:, :N]
    pooled = pooled_pad[:N]
    id_enc = _idenc_tc(walks_pad)[:, :, :N]
    return pooled, walks, id_enc
